# baseline (speedup 1.0000x reference)
import jax
import jax.numpy as jnp
from jax import lax
from jax.experimental import pallas as pl
from jax.experimental.pallas import tpu as pltpu
from jax.experimental.pallas import tpu_sc as plsc

NUM_CORES = 2
NUM_SUBCORES = 16
NUM_WORKERS = NUM_CORES * NUM_SUBCORES
LANES = 16
CHUNK = 128


def _max_abs_body(w_ref, o_ref):
    i = pl.program_id(0)
    m = jnp.max(jnp.abs(w_ref[...]))

    @pl.when(i == 0)
    def _():
        o_ref[0, 0] = m

    @pl.when(i != 0)
    def _():
        o_ref[0, 0] = jnp.maximum(o_ref[0, 0], m)


def _max_abs(weight):
    n = weight.size
    w2 = weight.reshape(n // 128, 128)
    rows = w2.shape[0]
    grid = 125
    blk = rows // grid
    assert blk * grid == rows
    return pl.pallas_call(
        _max_abs_body,
        grid=(grid,),
        in_specs=[pl.BlockSpec((blk, 128), lambda i: (i, 0))],
        out_specs=pl.BlockSpec(memory_space=pltpu.SMEM),
        out_shape=jax.ShapeDtypeStruct((1, 1), jnp.float32),
    )(w2)


def _gather_quant_body(idx_hbm, table_hbm, maxv_hbm, out_hbm,
                       idx_v, rows_v, maxv_v, sem):
    d = rows_v.shape[1]
    n_chunks = idx_v.shape[0]
    wid = lax.axis_index("s") * NUM_CORES + lax.axis_index("c")
    base = wid * (n_chunks * CHUNK)

    pltpu.sync_copy(idx_hbm.at[wid], idx_v)
    pltpu.sync_copy(maxv_hbm, maxv_v)
    vmax = maxv_v[...]
    vneg = -vmax
    vthr = vmax * (2.0 ** -24)

    def chunk_body(j, carry):
        pltpu.async_copy(table_hbm.at[idx_v.at[j]], rows_v, sem).wait()

        def row_body(r, carry2):
            for c in range(d // LANES):
                w = rows_v[r, pl.ds(c * LANES, LANES)]
                rows_v[r, pl.ds(c * LANES, LANES)] = jnp.where(
                    w > vthr, vmax, vneg)
            return carry2

        lax.fori_loop(0, CHUNK, row_body, 0)
        pltpu.sync_copy(rows_v, out_hbm.at[pl.ds(base + j * CHUNK, CHUNK)])
        return carry

    lax.fori_loop(0, n_chunks, chunk_body, 0)


def _gather_quant(idx3, weight, maxvec):
    n_chunks = idx3.shape[1]
    total = idx3.shape[0] * n_chunks * CHUNK
    d = weight.shape[1]
    mesh = plsc.VectorSubcoreMesh(core_axis_name="c", subcore_axis_name="s")
    f = pl.kernel(
        _gather_quant_body,
        out_type=jax.ShapeDtypeStruct((total, d), jnp.float32),
        mesh=mesh,
        scratch_types=[
            pltpu.VMEM((n_chunks, CHUNK), jnp.int32),
            pltpu.VMEM((CHUNK, d), jnp.float32),
            pltpu.VMEM((LANES,), jnp.float32),
            pltpu.SemaphoreType.DMA,
        ],
        compiler_params=pltpu.CompilerParams(use_tc_tiling_on_sc=False),
    )
    return f(idx3, weight, maxvec)


def kernel(input, weight):
    b, s = input.shape
    total = b * s
    per_worker = total // NUM_WORKERS
    n_chunks = per_worker // CHUNK
    assert n_chunks * CHUNK * NUM_WORKERS == total

    idx3 = input.astype(jnp.int32).reshape(NUM_WORKERS, n_chunks, CHUNK)
    maxv = _max_abs(weight)
    maxvec = jnp.broadcast_to(maxv.reshape(()), (LANES,))
    out = _gather_quant(idx3, weight, maxvec)
    return out.reshape(b, s, weight.shape[1])


# speedup vs baseline: 1.1580x; 1.1580x over previous
import jax
import jax.numpy as jnp
from jax import lax
from jax.experimental import pallas as pl
from jax.experimental.pallas import tpu as pltpu
from jax.experimental.pallas import tpu_sc as plsc

NUM_CORES = 2
NUM_SUBCORES = 16
NUM_WORKERS = NUM_CORES * NUM_SUBCORES
LANES = 16
CHUNK = 128


def _max_abs_body(w_ref, o_ref):
    i = pl.program_id(0)
    m = jnp.max(jnp.abs(w_ref[...]))

    @pl.when(i == 0)
    def _():
        o_ref[0, 0] = m

    @pl.when(i != 0)
    def _():
        o_ref[0, 0] = jnp.maximum(o_ref[0, 0], m)


def _max_abs(weight):
    rows, d = weight.shape
    grid = 125
    blk = rows // grid
    assert blk * grid == rows
    return pl.pallas_call(
        _max_abs_body,
        grid=(grid,),
        in_specs=[pl.BlockSpec((blk, d), lambda i: (i, 0))],
        out_specs=pl.BlockSpec(memory_space=pltpu.SMEM),
        out_shape=jax.ShapeDtypeStruct((1, 1), jnp.float32),
    )(weight)


def _gather_quant_body(idx_hbm, table_hbm, maxv_hbm, out_hbm,
                       idx_v, rows_v, maxv_v, sem):
    d = rows_v.shape[1]
    n_chunks = idx_v.shape[0]
    wid = lax.axis_index("s") * NUM_CORES + lax.axis_index("c")
    base = wid * (n_chunks * CHUNK)

    pltpu.sync_copy(idx_hbm.at[wid], idx_v)
    pltpu.sync_copy(maxv_hbm, maxv_v)
    vmax = maxv_v[...]
    vneg = -vmax
    vthr = vmax * (2.0 ** -24)

    def chunk_body(j, carry):
        pltpu.async_copy(table_hbm.at[idx_v.at[j]], rows_v, sem).wait()

        def row_body(r, carry2):
            for c in range(d // LANES):
                w = rows_v[r, pl.ds(c * LANES, LANES)]
                rows_v[r, pl.ds(c * LANES, LANES)] = jnp.where(
                    w > vthr, vmax, vneg)
            return carry2

        lax.fori_loop(0, CHUNK, row_body, 0)
        pltpu.sync_copy(rows_v, out_hbm.at[pl.ds(base + j * CHUNK, CHUNK)])
        return carry

    lax.fori_loop(0, n_chunks, chunk_body, 0)


def _gather_quant(idx3, weight, maxvec):
    n_chunks = idx3.shape[1]
    total = idx3.shape[0] * n_chunks * CHUNK
    d = weight.shape[1]
    mesh = plsc.VectorSubcoreMesh(core_axis_name="c", subcore_axis_name="s")
    f = pl.kernel(
        _gather_quant_body,
        out_type=jax.ShapeDtypeStruct((total, d), jnp.float32),
        mesh=mesh,
        scratch_types=[
            pltpu.VMEM((n_chunks, CHUNK), jnp.int32),
            pltpu.VMEM((CHUNK, d), jnp.float32),
            pltpu.VMEM((LANES,), jnp.float32),
            pltpu.SemaphoreType.DMA,
        ],
        compiler_params=pltpu.CompilerParams(use_tc_tiling_on_sc=False),
    )
    return f(idx3, weight, maxvec)


def kernel(input, weight):
    b, s = input.shape
    total = b * s
    per_worker = total // NUM_WORKERS
    n_chunks = per_worker // CHUNK
    assert n_chunks * CHUNK * NUM_WORKERS == total

    idx3 = input.astype(jnp.int32).reshape(NUM_WORKERS, n_chunks, CHUNK)
    maxv = _max_abs(weight)
    maxvec = jnp.broadcast_to(maxv.reshape(()), (LANES,))
    out = _gather_quant(idx3, weight, maxvec)
    return out.reshape(b, s, weight.shape[1])


# speedup vs baseline: 1.3221x; 1.1417x over previous
import jax
import jax.numpy as jnp
from jax import lax
from jax.experimental import pallas as pl
from jax.experimental.pallas import tpu as pltpu
from jax.experimental.pallas import tpu_sc as plsc

NUM_CORES = 2
NUM_SUBCORES = 16
NUM_WORKERS = NUM_CORES * NUM_SUBCORES
LANES = 16
CHUNK = 128


def _max_repack_body(w_ref, o_ref, t_ref):
    i = pl.program_id(0)
    x = w_ref[...]
    t_ref[...] = jnp.concatenate([x, x], axis=1)
    m = jnp.max(jnp.abs(x))

    @pl.when(i == 0)
    def _():
        o_ref[0, 0] = m

    @pl.when(i != 0)
    def _():
        o_ref[0, 0] = jnp.maximum(o_ref[0, 0], m)


def _max_repack(weight):
    rows, d = weight.shape
    grid = 125
    blk = rows // grid
    assert blk * grid == rows
    return pl.pallas_call(
        _max_repack_body,
        grid=(grid,),
        in_specs=[pl.BlockSpec((blk, d), lambda i: (i, 0))],
        out_specs=[
            pl.BlockSpec(memory_space=pltpu.SMEM),
            pl.BlockSpec((blk, 2 * d), lambda i: (i, 0)),
        ],
        out_shape=[
            jax.ShapeDtypeStruct((1, 1), jnp.float32),
            jax.ShapeDtypeStruct((rows, 2 * d), jnp.float32),
        ],
    )(weight)


def _gather_quant_body(idx_hbm, table_hbm, maxv_hbm, out_hbm,
                       idx_v, rows_v, out_v, maxv_v, sem):
    d = out_hbm.shape[1] // 2
    n_chunks = 50
    sup_per_chunk = CHUNK // 2
    wid = lax.axis_index("s") * NUM_CORES + lax.axis_index("c")
    base = wid * (n_chunks * sup_per_chunk)

    pltpu.sync_copy(idx_hbm.at[wid], idx_v)
    pltpu.sync_copy(maxv_hbm, maxv_v)
    vmax = maxv_v[...]
    vneg = -vmax
    vthr = vmax * (2.0 ** -24)

    def chunk_body(j, carry):
        pltpu.async_copy(table_hbm.at[idx_v.at[j]], rows_v, sem).wait()

        def pack_body(k, carry2):
            for half in range(2):
                r = 2 * k + half
                for c in range(d // LANES):
                    w = rows_v[r, pl.ds(c * LANES, LANES)]
                    out_v[k, pl.ds(half * d + c * LANES, LANES)] = jnp.where(
                        w > vthr, vmax, vneg)
            return carry2

        lax.fori_loop(0, sup_per_chunk, pack_body, 0)
        pltpu.sync_copy(
            out_v, out_hbm.at[pl.ds(base + j * sup_per_chunk, sup_per_chunk)])
        return carry

    lax.fori_loop(0, n_chunks, chunk_body, 0)


def _gather_quant(idxp, table, maxvec, total):
    d2 = table.shape[1]
    mesh = plsc.VectorSubcoreMesh(core_axis_name="c", subcore_axis_name="s")
    f = pl.kernel(
        _gather_quant_body,
        out_type=jax.ShapeDtypeStruct((total // 2, d2), jnp.float32),
        mesh=mesh,
        scratch_types=[
            pltpu.VMEM(idxp.shape[1:], jnp.int32),
            pltpu.VMEM((CHUNK, d2), jnp.float32),
            pltpu.VMEM((CHUNK // 2, d2), jnp.float32),
            pltpu.VMEM((LANES,), jnp.float32),
            pltpu.SemaphoreType.DMA,
        ],
        compiler_params=pltpu.CompilerParams(use_tc_tiling_on_sc=True),
    )
    return f(idxp, table, maxvec)


def kernel(input, weight):
    b, s = input.shape
    total = b * s
    per_worker = total // NUM_WORKERS
    n_chunks = per_worker // CHUNK
    assert n_chunks * CHUNK * NUM_WORKERS == total

    idx3 = input.astype(jnp.int32).reshape(NUM_WORKERS, n_chunks, CHUNK)
    idxp = jnp.pad(idx3, ((0, 0), (0, 56 - n_chunks), (0, 0)))
    maxv, packed = _max_repack(weight)
    maxvec = jnp.broadcast_to(maxv.reshape(()), (LANES,))
    out = _gather_quant(idxp, packed, maxvec, total)
    return out.reshape(b, s, weight.shape[1])
